# baseline (speedup 1.0000x reference)
import functools

import jax
import jax.numpy as jnp
from jax import lax
from jax.experimental import pallas as pl
from jax.experimental.pallas import tpu as pltpu
from jax.experimental.pallas import tpu_sc as plsc

_PAD = 0
_EOS = 1

_NC = 2
_NS = 16
_NW = _NC * _NS

_NBUF = 8
_L16 = 16


@functools.cache
def _make_kernel(b: int, l: int, table_rows: int, d: int):
    assert b % _NW == 0
    rows_per_w = b // _NW
    n_chunks = rows_per_w
    assert n_chunks % _NBUF == 0 and n_chunks >= 2 * _NBUF
    assert l >= _L16 and l % 8 == 0

    cols = list(range(0, l - _L16 + 1, _L16))
    if cols[-1] + _L16 < l:
        cols.append(l - _L16)

    mesh = plsc.VectorSubcoreMesh(core_axis_name="c", subcore_axis_name="s")

    @functools.partial(
        pl.kernel,
        mesh=mesh,
        out_type=(
            jax.ShapeDtypeStruct((b, l, d), jnp.float32),
            jax.ShapeDtypeStruct((b, l), jnp.float32),
            jax.ShapeDtypeStruct((b, l), jnp.float32),
        ),
        compiler_params=pltpu.CompilerParams(use_tc_tiling_on_sc=False),
        scratch_types=[
            pltpu.VMEM((_NBUF, l), jnp.int32),
            pltpu.VMEM((_NBUF, l, d), jnp.float32),
            pltpu.VMEM((_NBUF, l), jnp.float32),
            pltpu.VMEM((_NBUF, l), jnp.float32),
        ] + [pltpu.SemaphoreType.DMA] * (2 * _NBUF),
    )
    def fused_fn(table_hbm, ids_hbm, mat_hbm, real_hbm, eos_hbm,
                 idc_v, rows_v, real_v, eos_v, *sems):
        gsems = sems[:_NBUF]
        ssems = sems[_NBUF:]

        wid = lax.axis_index("s") * _NC + lax.axis_index("c")
        brow0 = wid * rows_per_w

        def i_copy(t, bb):
            return pltpu.make_async_copy(
                ids_hbm.at[brow0 + t], idc_v.at[bb], gsems[bb])

        def g_copy(t, bb):
            return pltpu.make_async_copy(
                table_hbm.at[idc_v.at[bb]], rows_v.at[bb], gsems[bb])

        def s_copy(t, bb):
            return pltpu.make_async_copy(
                rows_v.at[bb], mat_hbm.at[brow0 + t], ssems[bb])

        def r_copy(t, bb):
            return pltpu.make_async_copy(
                real_v.at[bb], real_hbm.at[brow0 + t], ssems[bb])

        def e_copy(t, bb):
            return pltpu.make_async_copy(
                eos_v.at[bb], eos_hbm.at[brow0 + t], ssems[bb])

        def masks(bb):
            for c in cols:
                v = idc_v[bb, pl.ds(c, _L16)]
                one = jnp.full((_L16,), 1.0, jnp.float32)
                zero = jnp.full((_L16,), 0.0, jnp.float32)
                real_v[bb, pl.ds(c, _L16)] = jnp.where(v != _PAD, one, zero)
                eos_v[bb, pl.ds(c, _L16)] = jnp.where(v == _EOS, one, zero)

        def fetch(t, bb):
            ic = i_copy(t, bb)
            ic.start()
            ic.wait()
            g_copy(t, bb).start()

        def finish(t, bb):
            g_copy(t, bb).wait()
            masks(bb)
            s_copy(t, bb).start()
            r_copy(t, bb).start()
            e_copy(t, bb).start()
            s_copy(t, bb).wait()
            r_copy(t, bb).wait()
            e_copy(t, bb).wait()

        for bb in range(_NBUF):
            fetch(bb, bb)

        @pl.loop(0, n_chunks - _NBUF, step=_NBUF)
        def _(t0):
            for bb in range(_NBUF):
                t = t0 + bb
                finish(t, bb)
                fetch(t + _NBUF, bb)

        for bb in range(_NBUF):
            finish(n_chunks - _NBUF + bb, bb)

    return fused_fn


def kernel(char_embedding, lookup_ids):
    b, l = lookup_ids.shape
    table_rows, d = char_embedding.shape
    ids32 = lookup_ids.astype(jnp.int32)
    mat, real, eos = _make_kernel(b, l, table_rows, d)(char_embedding, ids32)
    return mat, real, eos


# speedup vs baseline: 1.0132x; 1.0132x over previous
import functools

import jax
import jax.numpy as jnp
from jax import lax
from jax.experimental import pallas as pl
from jax.experimental.pallas import tpu as pltpu
from jax.experimental.pallas import tpu_sc as plsc

_PAD = 0
_EOS = 1

_NC = 2
_NS = 16
_NW = _NC * _NS

_CH = 512
_NBUF = 2
_L16 = 16


@functools.cache
def _make_kernel(n: int, table_rows: int, d: int):
    assert n % (_NW * _CH) == 0
    per_w = n // _NW
    n_chunks = per_w // _CH
    assert n_chunks % _NBUF == 0 and n_chunks >= 2 * _NBUF
    assert _CH % _L16 == 0

    mesh = plsc.VectorSubcoreMesh(core_axis_name="c", subcore_axis_name="s")

    @functools.partial(
        pl.kernel,
        mesh=mesh,
        out_type=(
            jax.ShapeDtypeStruct((n, d), jnp.float32),
            jax.ShapeDtypeStruct((n,), jnp.float32),
            jax.ShapeDtypeStruct((n,), jnp.float32),
        ),
        compiler_params=pltpu.CompilerParams(use_tc_tiling_on_sc=False),
        scratch_types=[
            pltpu.VMEM((_NBUF, _CH), jnp.int32),
            pltpu.VMEM((_NBUF, _CH, d), jnp.float32),
            pltpu.VMEM((_NBUF, _CH), jnp.float32),
            pltpu.VMEM((_NBUF, _CH), jnp.float32),
        ] + [pltpu.SemaphoreType.DMA] * (2 * _NBUF),
    )
    def fused_fn(table_hbm, ids_hbm, mat_hbm, real_hbm, eos_hbm,
                 idc_v, rows_v, real_v, eos_v, *sems):
        gsems = sems[:_NBUF]
        ssems = sems[_NBUF:]

        wid = lax.axis_index("s") * _NC + lax.axis_index("c")
        base = wid * per_w

        def i_copy(t, bb):
            return pltpu.make_async_copy(
                ids_hbm.at[pl.ds(base + t * _CH, _CH)], idc_v.at[bb],
                gsems[bb])

        def g_copy(t, bb):
            return pltpu.make_async_copy(
                table_hbm.at[idc_v.at[bb]], rows_v.at[bb], gsems[bb])

        def s_copy(t, bb):
            return pltpu.make_async_copy(
                rows_v.at[bb], mat_hbm.at[pl.ds(base + t * _CH, _CH)],
                ssems[bb])

        def r_copy(t, bb):
            return pltpu.make_async_copy(
                real_v.at[bb], real_hbm.at[pl.ds(base + t * _CH, _CH)],
                ssems[bb])

        def e_copy(t, bb):
            return pltpu.make_async_copy(
                eos_v.at[bb], eos_hbm.at[pl.ds(base + t * _CH, _CH)],
                ssems[bb])

        def masks(bb):
            one = jnp.full((_L16,), 1.0, jnp.float32)
            zero = jnp.full((_L16,), 0.0, jnp.float32)
            for c in range(0, _CH, _L16):
                v = idc_v[bb, pl.ds(c, _L16)]
                real_v[bb, pl.ds(c, _L16)] = jnp.where(v != _PAD, one, zero)
                eos_v[bb, pl.ds(c, _L16)] = jnp.where(v == _EOS, one, zero)

        def fetch(t, bb):
            ic = i_copy(t, bb)
            ic.start()
            ic.wait()
            g_copy(t, bb).start()

        def finish(t, bb):
            g_copy(t, bb).wait()
            masks(bb)
            s_copy(t, bb).start()
            r_copy(t, bb).start()
            e_copy(t, bb).start()
            s_copy(t, bb).wait()
            r_copy(t, bb).wait()
            e_copy(t, bb).wait()

        for bb in range(_NBUF):
            fetch(bb, bb)

        @pl.loop(0, n_chunks - _NBUF, step=_NBUF)
        def _(t0):
            for bb in range(_NBUF):
                t = t0 + bb
                finish(t, bb)
                fetch(t + _NBUF, bb)

        for bb in range(_NBUF):
            finish(n_chunks - _NBUF + bb, bb)

    return fused_fn


def kernel(char_embedding, lookup_ids):
    b, l = lookup_ids.shape
    table_rows, d = char_embedding.shape
    flat_ids = lookup_ids.astype(jnp.int32).reshape(-1)
    mat, real, eos = _make_kernel(b * l, table_rows, d)(
        char_embedding, flat_ids)
    return mat.reshape(b, l, d), real.reshape(b, l), eos.reshape(b, l)


# speedup vs baseline: 1.0264x; 1.0130x over previous
import functools

import jax
import jax.numpy as jnp
from jax import lax
from jax.experimental import pallas as pl
from jax.experimental.pallas import tpu as pltpu
from jax.experimental.pallas import tpu_sc as plsc

_PAD = 0
_EOS = 1

_NC = 2
_NS = 16
_NW = _NC * _NS

_CH = 512
_NBUF = 2
_L16 = 16


@functools.cache
def _make_kernel(n: int, table_rows: int, d: int):
    assert n % (_NW * _CH) == 0
    per_w = n // _NW
    n_chunks = per_w // _CH
    assert n_chunks % _NBUF == 0 and n_chunks >= 2 * _NBUF
    assert _CH % _L16 == 0

    mesh = plsc.VectorSubcoreMesh(core_axis_name="c", subcore_axis_name="s")

    @functools.partial(
        pl.kernel,
        mesh=mesh,
        out_type=(
            jax.ShapeDtypeStruct((n, d), jnp.float32),
            jax.ShapeDtypeStruct((n,), jnp.float32),
            jax.ShapeDtypeStruct((n,), jnp.float32),
        ),
        compiler_params=pltpu.CompilerParams(use_tc_tiling_on_sc=False),
        scratch_types=[
            pltpu.VMEM((per_w,), jnp.int32),
            pltpu.VMEM((_NBUF, _CH, d), jnp.float32),
            pltpu.VMEM((_NBUF, _CH), jnp.float32),
            pltpu.VMEM((_NBUF, _CH), jnp.float32),
        ] + [pltpu.SemaphoreType.DMA] * (2 * _NBUF),
    )
    def fused_fn(table_hbm, ids_hbm, mat_hbm, real_hbm, eos_hbm,
                 idx_v, rows_v, real_v, eos_v, *sems):
        gsems = sems[:_NBUF]
        ssems = sems[_NBUF:]

        wid = lax.axis_index("s") * _NC + lax.axis_index("c")
        base = wid * per_w
        pltpu.sync_copy(ids_hbm.at[pl.ds(base, per_w)], idx_v)

        def g_copy(t, bb):
            return pltpu.make_async_copy(
                table_hbm.at[idx_v.at[pl.ds(t * _CH, _CH)]], rows_v.at[bb],
                gsems[bb])

        def s_copy(t, bb):
            return pltpu.make_async_copy(
                rows_v.at[bb], mat_hbm.at[pl.ds(base + t * _CH, _CH)],
                ssems[bb])

        def r_copy(t, bb):
            return pltpu.make_async_copy(
                real_v.at[bb], real_hbm.at[pl.ds(base + t * _CH, _CH)],
                ssems[bb])

        def e_copy(t, bb):
            return pltpu.make_async_copy(
                eos_v.at[bb], eos_hbm.at[pl.ds(base + t * _CH, _CH)],
                ssems[bb])

        def masks(t, bb):
            one = jnp.full((_L16,), 1.0, jnp.float32)
            zero = jnp.full((_L16,), 0.0, jnp.float32)
            for c in range(0, _CH, _L16):
                v = idx_v[pl.ds(t * _CH + c, _L16)]
                real_v[bb, pl.ds(c, _L16)] = jnp.where(v != _PAD, one, zero)
                eos_v[bb, pl.ds(c, _L16)] = jnp.where(v == _EOS, one, zero)

        def fetch(t, bb):
            g_copy(t, bb).start()
            masks(t, bb)

        def finish(t, bb):
            g_copy(t, bb).wait()
            s_copy(t, bb).start()
            r_copy(t, bb).start()
            e_copy(t, bb).start()
            s_copy(t, bb).wait()
            r_copy(t, bb).wait()
            e_copy(t, bb).wait()

        for bb in range(_NBUF):
            fetch(bb, bb)

        @pl.loop(0, n_chunks - _NBUF, step=_NBUF)
        def _(t0):
            for bb in range(_NBUF):
                t = t0 + bb
                finish(t, bb)
                fetch(t + _NBUF, bb)

        for bb in range(_NBUF):
            finish(n_chunks - _NBUF + bb, bb)

    return fused_fn


def kernel(char_embedding, lookup_ids):
    b, l = lookup_ids.shape
    table_rows, d = char_embedding.shape
    flat_ids = lookup_ids.astype(jnp.int32).reshape(-1)
    mat, real, eos = _make_kernel(b * l, table_rows, d)(
        char_embedding, flat_ids)
    return mat.reshape(b, l, d), real.reshape(b, l), eos.reshape(b, l)


# speedup vs baseline: 1.3603x; 1.3253x over previous
import functools

import jax
import jax.numpy as jnp
from jax import lax
from jax.experimental import pallas as pl
from jax.experimental.pallas import tpu as pltpu
from jax.experimental.pallas import tpu_sc as plsc

_PAD = 0
_EOS = 1

_NC = 2
_NS = 16
_NW = _NC * _NS

_CH = 512
_NBUF = 2
_L16 = 16


@functools.cache
def _make_kernel(n: int, table_rows: int, d: int):
    assert n % (_NW * _CH) == 0
    per_w = n // _NW
    n_chunks = per_w // _CH
    assert n_chunks % _NBUF == 0 and n_chunks >= 2 * _NBUF
    assert _CH % _L16 == 0

    mesh = plsc.VectorSubcoreMesh(core_axis_name="c", subcore_axis_name="s")

    @functools.partial(
        pl.kernel,
        mesh=mesh,
        out_type=(
            jax.ShapeDtypeStruct((n, 128), jnp.float32),
            jax.ShapeDtypeStruct((n,), jnp.float32),
            jax.ShapeDtypeStruct((n,), jnp.float32),
        ),
        compiler_params=pltpu.CompilerParams(use_tc_tiling_on_sc=False),
        scratch_types=[
            pltpu.VMEM((per_w,), jnp.int32),
            pltpu.VMEM((_NBUF, _CH, d), jnp.float32),
            pltpu.VMEM((_NBUF, _CH), jnp.float32),
            pltpu.VMEM((_NBUF, _CH), jnp.float32),
        ] + [pltpu.SemaphoreType.DMA] * (2 * _NBUF),
    )
    def fused_fn(table_hbm, ids_hbm, mat_hbm, real_hbm, eos_hbm,
                 idx_v, rows_v, real_v, eos_v, *sems):
        gsems = sems[:_NBUF]
        ssems = sems[_NBUF:]

        wid = lax.axis_index("s") * _NC + lax.axis_index("c")
        base = wid * per_w
        pltpu.sync_copy(ids_hbm.at[pl.ds(base, per_w)], idx_v)

        def g_copy(t, bb):
            return pltpu.make_async_copy(
                table_hbm.at[idx_v.at[pl.ds(t * _CH, _CH)]], rows_v.at[bb],
                gsems[bb])

        def s_copy(t, bb):
            return pltpu.make_async_copy(
                rows_v.at[bb],
                mat_hbm.at[pl.ds(base + t * _CH, _CH), pl.ds(0, d)],
                ssems[bb])

        def r_copy(t, bb):
            return pltpu.make_async_copy(
                real_v.at[bb], real_hbm.at[pl.ds(base + t * _CH, _CH)],
                ssems[bb])

        def e_copy(t, bb):
            return pltpu.make_async_copy(
                eos_v.at[bb], eos_hbm.at[pl.ds(base + t * _CH, _CH)],
                ssems[bb])

        def masks(t, bb):
            one = jnp.full((_L16,), 1.0, jnp.float32)
            zero = jnp.full((_L16,), 0.0, jnp.float32)
            for c in range(0, _CH, _L16):
                v = idx_v[pl.ds(t * _CH + c, _L16)]
                real_v[bb, pl.ds(c, _L16)] = jnp.where(v != _PAD, one, zero)
                eos_v[bb, pl.ds(c, _L16)] = jnp.where(v == _EOS, one, zero)

        def fetch(t, bb):
            g_copy(t, bb).start()
            masks(t, bb)

        def finish(t, bb):
            g_copy(t, bb).wait()
            s_copy(t, bb).start()
            r_copy(t, bb).start()
            e_copy(t, bb).start()
            s_copy(t, bb).wait()
            r_copy(t, bb).wait()
            e_copy(t, bb).wait()

        for bb in range(_NBUF):
            fetch(bb, bb)

        @pl.loop(0, n_chunks - _NBUF, step=_NBUF)
        def _(t0):
            for bb in range(_NBUF):
                t = t0 + bb
                finish(t, bb)
                fetch(t + _NBUF, bb)

        for bb in range(_NBUF):
            finish(n_chunks - _NBUF + bb, bb)

    return fused_fn


def kernel(char_embedding, lookup_ids):
    b, l = lookup_ids.shape
    table_rows, d = char_embedding.shape
    flat_ids = lookup_ids.astype(jnp.int32).reshape(-1)
    mat, real, eos = _make_kernel(b * l, table_rows, d)(
        char_embedding, flat_ids)
    return mat[:, :d].reshape(b, l, d), real.reshape(b, l), eos.reshape(b, l)
